# baseline (speedup 1.0000x reference)
import functools

import jax
import jax.numpy as jnp
from jax import lax
from jax.experimental import pallas as pl
from jax.experimental.pallas import tpu as pltpu
from jax.experimental.pallas import tpu_sc as plsc

B = 256
T = 16
F = 16080
NBUF = 4

_mesh = plsc.VectorSubcoreMesh(core_axis_name="c", subcore_axis_name="s")


@functools.partial(
    pl.kernel,
    out_type=[
        jax.ShapeDtypeStruct((B, T, F), jnp.float32),
        jax.ShapeDtypeStruct((T, F), jnp.float32),
    ],
    mesh=_mesh,
    compiler_params=pltpu.CompilerParams(use_tc_tiling_on_sc=True),
    scratch_types=[
        [pltpu.VMEM((F,), jnp.float32) for _ in range(NBUF)],
        pltpu.SemaphoreType.DMA((NBUF,)),
    ],
)
def _zfill(out_hbm, att_hbm, bufs, sems):
    c = lax.axis_index("c")
    s = lax.axis_index("s")
    w = s * 2 + c

    def zinit(j, _):
        for buf in bufs:
            buf[pl.ds(j * 16, 16)] = jnp.zeros((16,), jnp.float32)
        return 0

    lax.fori_loop(0, F // 16, zinit, 0)

    rows_per_w = (B * T) // 32

    def body(i0, _):
        for k in range(NBUF):
            idx = w * rows_per_w + i0 * NBUF + k
            b = idx // T
            t = idx % T
            cp = pltpu.make_async_copy(bufs[k], out_hbm.at[b, t], sems.at[k])

            @pl.when(i0 > 0)
            def _():
                pltpu.make_async_copy(
                    bufs[k], out_hbm.at[b, t], sems.at[k]
                ).wait()

            cp.start()
        return 0

    lax.fori_loop(0, rows_per_w // NBUF, body, 0)

    for k in range(NBUF):
        pltpu.make_async_copy(bufs[k], out_hbm.at[0, k], sems.at[k]).wait()

    @pl.when(c == 0)
    def _():
        cp = pltpu.make_async_copy(bufs[0], att_hbm.at[s], sems.at[0])
        cp.start()
        cp.wait()


def kernel(x, attention_mask):
    del x, attention_mask
    out, att = _zfill()
    return out, att


# speedup vs baseline: 3.5955x; 3.5955x over previous
import jax
import jax.numpy as jnp
from jax.experimental import pallas as pl
from jax.experimental.pallas import tpu as pltpu

B = 256
T = 16
F = 16080
C = 1000
CP = 1024
K = 200

ZF = 1024
NZQ = 8
NHQ = 2


def _attention_values(mask):
    s = jax.nn.sigmoid(mask)
    bits = jax.lax.bitcast_convert_type(s, jnp.int32)

    def bstep(_, lohi):
        lo, hi = lohi
        mid = lo + (hi - lo + 1) // 2
        cnt = jnp.sum((bits >= mid).astype(jnp.int32), axis=1, keepdims=True)
        ge = cnt >= K
        return jnp.where(ge, mid, lo), jnp.where(ge, hi, mid - 1)

    lo0 = jnp.zeros((T, 1), jnp.int32)
    hi0 = jnp.full((T, 1), 0x3F800000, jnp.int32)
    thr, _ = jax.lax.fori_loop(0, 31, bstep, (lo0, hi0))

    col = jax.lax.broadcasted_iota(jnp.int32, (T, C), 1)
    gt = bits > thr
    eq = bits == thr
    need = K - jnp.sum(gt.astype(jnp.int32), axis=1, keepdims=True)

    def istep(_, lohi):
        lo, hi = lohi
        mid = (lo + hi) // 2
        cnt = jnp.sum((eq & (col < mid)).astype(jnp.int32), axis=1,
                      keepdims=True)
        ok = cnt >= need
        return jnp.where(ok, lo, mid + 1), jnp.where(ok, mid, hi)

    plo0 = jnp.zeros((T, 1), jnp.int32)
    phi0 = jnp.full((T, 1), C, jnp.int32)
    pcut, _ = jax.lax.fori_loop(0, 10, istep, (plo0, phi0))

    keep = gt | (eq & (col < pcut))
    return jnp.where(keep, s, 0.0)


def _body(mask_ref, xt_ref, out_ref, att_ref, zbuf, hbufs, zsems, hsems):
    att = _attention_values(mask_ref[...])
    att_ref[:, :C] = att
    att_ref[:, C:] = jnp.zeros((T, F - C), jnp.float32)
    attp = jnp.concatenate(
        [att, jnp.zeros((T, CP - C), jnp.float32)], axis=1)

    zbuf[...] = jnp.zeros((ZF, B), jnp.float32)

    zq = 0
    zwaits = []
    for t in range(T):
        f0 = CP
        while f0 < F:
            n = min(ZF, F - f0)
            cp = pltpu.make_async_copy(
                zbuf.at[pl.ds(0, n), :],
                out_ref.at[t, pl.ds(f0, n), :],
                zsems.at[zq % NZQ],
            )
            if len(zwaits) >= NZQ:
                zwaits.pop(0).wait()
            cp.start()
            zwaits.append(cp)
            f0 += n
            zq += 1

    def hcopy(t, buf):
        return pltpu.make_async_copy(
            buf, out_ref.at[t, pl.ds(0, CP), :], hsems.at[t % NHQ]
        )

    hprev = []
    for t in range(T):
        buf = hbufs[t % NHQ]
        if len(hprev) >= NHQ:
            hprev.pop(0).wait()
        buf[...] = xt_ref[...] * attp[t][:, None]
        cp = hcopy(t, buf)
        cp.start()
        hprev.append(cp)

    for cp in zwaits:
        cp.wait()
    for cp in hprev:
        cp.wait()


def kernel(x, attention_mask):
    xt = x[:, :6, :].reshape(B, 6 * 201)[:, :CP].T
    out_t, att = pl.pallas_call(
        _body,
        in_specs=[
            pl.BlockSpec(memory_space=pltpu.VMEM),
            pl.BlockSpec(memory_space=pltpu.VMEM),
        ],
        out_specs=[
            pl.BlockSpec(memory_space=pl.MemorySpace.ANY),
            pl.BlockSpec(memory_space=pltpu.VMEM),
        ],
        out_shape=[
            jax.ShapeDtypeStruct((T, F, B), jnp.float32),
            jax.ShapeDtypeStruct((T, F), jnp.float32),
        ],
        scratch_shapes=[
            pltpu.VMEM((ZF, B), jnp.float32),
            [pltpu.VMEM((CP, B), jnp.float32) for _ in range(NHQ)],
            pltpu.SemaphoreType.DMA((NZQ,)),
            pltpu.SemaphoreType.DMA((NHQ,)),
        ],
    )(attention_mask, xt)
    return jnp.transpose(out_t, (2, 0, 1)), att
